# baseline (speedup 1.0000x reference)
import functools

import jax
import jax.numpy as jnp
from jax import lax
from jax.experimental import pallas as pl
from jax.experimental.pallas import tpu as pltpu
from jax.experimental.pallas import tpu_sc as plsc

_BBOX_STD = (0.1, 0.1, 0.2, 0.2)
_MAX_OUT = 100
_NMS_THR = 0.3
_CONF = 0.15
_NEG = -1e30

_ROWS = 160
_LANES = 128
_P = _ROWS * _LANES
_NT = 32
_CH = _P // _NT
_NC = 16
_D = _CH // _NC
_SORT_N = 64
_BIGI = jnp.int32(2**30)


def _prep_body(boxes_ref, deltas_ref, probs_ref, meta_ref,
               ukey_ref, y1_ref, x1_ref, y2_ref, x2_ref):
    h = meta_ref[0, 4]
    w = meta_ref[0, 5]
    wy1 = (meta_ref[0, 7] - 0.0) / (h - 1.0)
    wx1 = (meta_ref[0, 8] - 0.0) / (w - 1.0)
    wy2 = (meta_ref[0, 9] - 1.0) / (h - 1.0)
    wx2 = (meta_ref[0, 10] - 1.0) / (w - 1.0)

    ry1 = boxes_ref[0]
    rx1 = boxes_ref[1]
    ry2 = boxes_ref[2]
    rx2 = boxes_ref[3]
    dy = deltas_ref[0] * _BBOX_STD[0]
    dx = deltas_ref[1] * _BBOX_STD[1]
    dh = deltas_ref[2] * _BBOX_STD[2]
    dw = deltas_ref[3] * _BBOX_STD[3]

    height = ry2 - ry1
    width = rx2 - rx1
    cy = ry1 + 0.5 * height + dy * height
    cx = rx1 + 0.5 * width + dx * width
    height = height * jnp.exp(dh)
    width = width * jnp.exp(dw)
    y1 = cy - 0.5 * height
    x1 = cx - 0.5 * width
    y2 = y1 + height
    x2 = x1 + width

    y1_ref[...] = jnp.clip(y1, wy1, wy2)
    x1_ref[...] = jnp.clip(x1, wx1, wx2)
    y2_ref[...] = jnp.clip(y2, wy1, wy2)
    x2_ref[...] = jnp.clip(x2, wx1, wx2)

    probs = probs_ref[...]
    bits = lax.bitcast_convert_type(probs, jnp.int32)
    ukey_ref[...] = jnp.where(probs >= _CONF, bits, jnp.int32(-1))


def _lane():
    return lax.broadcasted_iota(jnp.int32, (16,), 0)


def _sort_body(ukey_hbm, y1_hbm, x1_hbm, y2_hbm, x2_hbm,
               us_hbm, gs_hbm, sy1_hbm, sx1_hbm, sy2_hbm, sx2_hbm,
               uloc, y1l, x1l, y2l, x2l, key, gid,
               e_us, e_gs, e_y1, e_x1, e_y2, e_x2):
    tid = lax.axis_index("s") * 2 + lax.axis_index("c")
    base = tid * _CH
    LANE = _lane()

    pltpu.sync_copy(ukey_hbm.at[pl.ds(base, _CH)], uloc)
    pltpu.sync_copy(y1_hbm.at[pl.ds(base, _CH)], y1l)
    pltpu.sync_copy(x1_hbm.at[pl.ds(base, _CH)], x1l)
    pltpu.sync_copy(y2_hbm.at[pl.ds(base, _CH)], y2l)
    pltpu.sync_copy(x2_hbm.at[pl.ds(base, _CH)], x2l)

    for r in range(_SORT_N):
        if r < _D:
            kv = uloc[pl.ds(r * 16, 16)]
            gv = base + r * 16 + LANE
        else:
            kv = jnp.full((16,), -1, jnp.int32)
            gv = jnp.full((16,), _P, jnp.int32)
        key[pl.ds(r * 16, 16)] = kv
        gid[pl.ds(r * 16, 16)] = gv

    k = 2
    while k <= _SORT_N:
        j = k // 2
        while j >= 1:
            sh = j.bit_length() - 1

            def ce(i, _, j=j, k=k, sh=sh):
                a = ((i >> sh) << (sh + 1)) | (i & (j - 1))
                b = a | j
                ia = a * 16 + LANE
                ib = b * 16 + LANE
                ka = plsc.load_gather(key, [ia])
                kb = plsc.load_gather(key, [ib])
                ga = plsc.load_gather(gid, [ia])
                gb = plsc.load_gather(gid, [ib])
                agtb = (ka > kb) | ((ka == kb) & (ga < gb))
                kg = jnp.where(agtb, ka, kb)
                kl = jnp.where(agtb, kb, ka)
                gg = jnp.where(agtb, ga, gb)
                gl = jnp.where(agtb, gb, ga)
                desc = (a & k) == 0
                plsc.store_scatter(key, [ia], jnp.where(desc, kg, kl))
                plsc.store_scatter(key, [ib], jnp.where(desc, kl, kg))
                plsc.store_scatter(gid, [ia], jnp.where(desc, gg, gl))
                plsc.store_scatter(gid, [ib], jnp.where(desc, gl, gg))
                return 0

            lax.fori_loop(0, _SORT_N // 2, ce, 0)
            j //= 2
        k *= 2

    for c in range(_NC):
        for ch in range((_D + 15) // 16):
            dvec = ch * 16 + LANE
            msk = dvec < _D
            src = jnp.minimum(dvec, _SORT_N - 1) * 16 + c
            kv = plsc.load_gather(key, [src])
            gv = plsc.load_gather(gid, [src])
            lidx = jnp.clip(gv - base, 0, _CH - 1)
            dst = jnp.minimum(c * _D + dvec, _CH - 1)
            plsc.store_scatter(e_us, [dst], kv, mask=msk)
            plsc.store_scatter(e_gs, [dst], gv, mask=msk)
            plsc.store_scatter(e_y1, [dst], plsc.load_gather(y1l, [lidx]),
                               mask=msk)
            plsc.store_scatter(e_x1, [dst], plsc.load_gather(x1l, [lidx]),
                               mask=msk)
            plsc.store_scatter(e_y2, [dst], plsc.load_gather(y2l, [lidx]),
                               mask=msk)
            plsc.store_scatter(e_x2, [dst], plsc.load_gather(x2l, [lidx]),
                               mask=msk)

    pltpu.sync_copy(e_us, us_hbm.at[pl.ds(base, _CH)])
    pltpu.sync_copy(e_gs, gs_hbm.at[pl.ds(base, _CH)])
    pltpu.sync_copy(e_y1, sy1_hbm.at[pl.ds(base, _CH)])
    pltpu.sync_copy(e_x1, sx1_hbm.at[pl.ds(base, _CH)])
    pltpu.sync_copy(e_y2, sy2_hbm.at[pl.ds(base, _CH)])
    pltpu.sync_copy(e_x2, sx2_hbm.at[pl.ds(base, _CH)])


def _splat_i(x):
    return jnp.zeros((16,), jnp.int32) + x


def _splat_f(x):
    return jnp.zeros((16,), jnp.float32) + x


def _scan_body(us_hbm, gs_hbm, sy1_hbm, sx1_hbm, sy2_hbm, sx2_hbm,
               det_hbm,
               su, sg, sy1, sx1, sy2, sx2,
               head_u, head_g, head_d, grp_u, grp_g,
               v_y1, v_x1, v_y2, v_x2, v_ar, outb):
    wid = lax.axis_index("s") * 2 + lax.axis_index("c")
    LANE = _lane()
    NS = _NT * _NC

    @pl.when(wid == 0)
    def _():
        pltpu.sync_copy(us_hbm, su)
        pltpu.sync_copy(gs_hbm, sg)
        pltpu.sync_copy(sy1_hbm, sy1)
        pltpu.sync_copy(sx1_hbm, sx1)
        pltpu.sync_copy(sy2_hbm, sy2)
        pltpu.sync_copy(sx2_hbm, sx2)

        zf = jnp.zeros((16,), jnp.float32)
        for jj in range(7):
            v_y1[pl.ds(jj * 16, 16)] = zf
            v_x1[pl.ds(jj * 16, 16)] = zf
            v_y2[pl.ds(jj * 16, 16)] = zf
            v_x2[pl.ds(jj * 16, 16)] = zf
            v_ar[pl.ds(jj * 16, 16)] = zf
        for r in range(_MAX_OUT):
            outb[pl.ds(r * 16, 16)] = zf

        def init_heads(i, _):
            svec = i * 16 + LANE
            posv = svec * _D
            hu = plsc.load_gather(su, [posv])
            hg = plsc.load_gather(sg, [posv])
            plsc.store_scatter(head_u, [svec], hu)
            plsc.store_scatter(head_g, [svec], hg)
            plsc.store_scatter(head_d, [svec], jnp.zeros((16,), jnp.int32))
            gm = jnp.max(hu)
            gg = jnp.min(jnp.where(hu == gm, hg, _BIGI))
            m0 = LANE == 0
            plsc.store_scatter(grp_u, [_splat_i(i)], _splat_i(gm), mask=m0)
            plsc.store_scatter(grp_g, [_splat_i(i)], _splat_i(gg), mask=m0)
            return 0

        lax.fori_loop(0, NS // 16, init_heads, 0)

        def body(carry):
            count, _go = carry
            g0u = grp_u[pl.ds(0, 16)]
            g1u = grp_u[pl.ds(16, 16)]
            g0g = grp_g[pl.ds(0, 16)]
            g1g = grp_g[pl.ds(16, 16)]
            mu = jnp.maximum(jnp.max(g0u), jnp.max(g1u))
            valid = mu >= 0
            mg = jnp.minimum(
                jnp.min(jnp.where(g0u == mu, g0g, _BIGI)),
                jnp.min(jnp.where(g1u == mu, g1g, _BIGI)))
            h0 = (g0u == mu) & (g0g == mg)
            h1 = (g1u == mu) & (g1g == mg)
            g = jnp.minimum(
                jnp.min(jnp.where(h0, LANE, jnp.int32(99))),
                jnp.min(jnp.where(h1, LANE + 16, jnp.int32(99))))
            g = jnp.minimum(g, jnp.int32(31))
            hvec = g * 16 + LANE
            hu16 = plsc.load_gather(head_u, [hvec])
            hg16 = plsc.load_gather(head_g, [hvec])
            lsel = jnp.min(jnp.where((hu16 == mu) & (hg16 == mg), LANE,
                                     jnp.int32(15)))
            s = jnp.minimum(g * 16 + lsel, jnp.int32(NS - 1))
            d = jnp.max(plsc.load_gather(head_d, [_splat_i(s)]))
            pos = jnp.minimum(s * _D + d, jnp.int32(_P - 1))
            posv = _splat_i(pos)
            y1b = jnp.max(plsc.load_gather(sy1, [posv]))
            x1b = jnp.max(plsc.load_gather(sx1, [posv]))
            y2b = jnp.max(plsc.load_gather(sy2, [posv]))
            x2b = jnp.max(plsc.load_gather(sx2, [posv]))
            area_b = (y2b - y1b) * (x2b - x1b)

            def iou_blk(jblk, acc):
                svi = jblk * 16 + LANE
                ya = plsc.load_gather(v_y1, [svi])
                xa = plsc.load_gather(v_x1, [svi])
                yb2 = plsc.load_gather(v_y2, [svi])
                xb2 = plsc.load_gather(v_x2, [svi])
                ar = plsc.load_gather(v_ar, [svi])
                yy1 = jnp.maximum(ya, y1b)
                xx1 = jnp.maximum(xa, x1b)
                yy2 = jnp.minimum(yb2, y2b)
                xx2 = jnp.minimum(xb2, x2b)
                inter = (jnp.maximum(yy2 - yy1, 0.0)
                         * jnp.maximum(xx2 - xx1, 0.0))
                denom = ar + area_b - inter + 1e-9
                return jnp.maximum(acc, inter - _NMS_THR * denom)

            acc = lax.fori_loop(0, 7, iou_blk, jnp.full((16,), -1.0,
                                                        jnp.float32))
            sup = jnp.max(acc) > 0.0
            accept = valid & jnp.logical_not(sup)

            m0a = (LANE == 0) & accept
            cidx = _splat_i(count)
            plsc.store_scatter(v_y1, [cidx], _splat_f(y1b), mask=m0a)
            plsc.store_scatter(v_x1, [cidx], _splat_f(x1b), mask=m0a)
            plsc.store_scatter(v_y2, [cidx], _splat_f(y2b), mask=m0a)
            plsc.store_scatter(v_x2, [cidx], _splat_f(x2b), mask=m0a)
            plsc.store_scatter(v_ar, [cidx], _splat_f(area_b), mask=m0a)

            score = jnp.max(plsc.bitcast(_splat_i(mu), jnp.float32))
            row = jnp.where(
                LANE == 0, y1b,
                jnp.where(LANE == 1, x1b,
                          jnp.where(LANE == 2, y2b,
                                    jnp.where(LANE == 3, x2b,
                                              jnp.where(LANE == 4, score,
                                                        0.0)))))
            outm = (jnp.zeros((16,), jnp.bool_) | accept)
            plsc.store_scatter(outb, [count * 16 + LANE], row, mask=outm)
            count2 = count + jnp.where(accept, 1, 0).astype(jnp.int32)

            nd = d + 1
            ex = nd >= _D
            npos = _splat_i(jnp.minimum(pos + 1, jnp.int32(_P - 1)))
            nu = jnp.where(ex, jnp.int32(-1),
                           jnp.max(plsc.load_gather(su, [npos])))
            ng = jnp.where(ex, _BIGI, jnp.max(plsc.load_gather(sg, [npos])))
            m0v = (LANE == 0) & valid
            sv_ = _splat_i(s)
            plsc.store_scatter(head_u, [sv_], _splat_i(nu), mask=m0v)
            plsc.store_scatter(head_g, [sv_], _splat_i(ng), mask=m0v)
            plsc.store_scatter(head_d, [sv_], _splat_i(nd), mask=m0v)
            hu2 = plsc.load_gather(head_u, [hvec])
            hg2 = plsc.load_gather(head_g, [hvec])
            gm2 = jnp.max(hu2)
            gg2 = jnp.min(jnp.where(hu2 == gm2, hg2, _BIGI))
            gidxv = _splat_i(g)
            plsc.store_scatter(grp_u, [gidxv], _splat_i(gm2), mask=m0v)
            plsc.store_scatter(grp_g, [gidxv], _splat_i(gg2), mask=m0v)

            go2 = valid & (count2 < _MAX_OUT)
            return (count2, go2)

        lax.while_loop(lambda c: c[1], body,
                       (jnp.int32(0), jnp.bool_(True)))

        pltpu.sync_copy(outb, det_hbm)


@functools.partial(jax.jit, static_argnames=())
def kernel(rois, bbox_scores, macacnn_bbox, image_meta):
    B, N = rois.shape[0], rois.shape[1]
    pad = _P - N

    def prep(x, fill):
        xt = jnp.transpose(x, (1, 0))
        xt = jnp.pad(xt, ((0, 0), (0, pad)), constant_values=fill)
        return xt.reshape(x.shape[1], _ROWS, _LANES)

    boxes_in = prep(rois[0], 0.0)
    deltas_in = prep(macacnn_bbox[0], 0.0)
    probs_in = prep(bbox_scores[0], -1.0)[0]

    f32 = jnp.float32
    ukey2d, y1p, x1p, y2p, x2p = pl.pallas_call(
        _prep_body,
        out_shape=[
            jax.ShapeDtypeStruct((_ROWS, _LANES), jnp.int32),
            jax.ShapeDtypeStruct((_ROWS, _LANES), f32),
            jax.ShapeDtypeStruct((_ROWS, _LANES), f32),
            jax.ShapeDtypeStruct((_ROWS, _LANES), f32),
            jax.ShapeDtypeStruct((_ROWS, _LANES), f32),
        ],
        in_specs=[
            pl.BlockSpec(memory_space=pltpu.VMEM),
            pl.BlockSpec(memory_space=pltpu.VMEM),
            pl.BlockSpec(memory_space=pltpu.VMEM),
            pl.BlockSpec(memory_space=pltpu.SMEM),
        ],
        out_specs=[pl.BlockSpec(memory_space=pltpu.VMEM)] * 5,
    )(boxes_in, deltas_in, probs_in, image_meta)

    ukey = ukey2d.reshape(_P)
    y1f = y1p.reshape(_P)
    x1f = x1p.reshape(_P)
    y2f = y2p.reshape(_P)
    x2f = x2p.reshape(_P)

    mesh = plsc.VectorSubcoreMesh(core_axis_name="c", subcore_axis_name="s")

    sc_params = pltpu.CompilerParams(needs_layout_passes=False)

    sort_k = functools.partial(
        pl.kernel, mesh=mesh, compiler_params=sc_params,
        out_type=[
            jax.ShapeDtypeStruct((_P,), jnp.int32),
            jax.ShapeDtypeStruct((_P,), jnp.int32),
            jax.ShapeDtypeStruct((_P,), f32),
            jax.ShapeDtypeStruct((_P,), f32),
            jax.ShapeDtypeStruct((_P,), f32),
            jax.ShapeDtypeStruct((_P,), f32),
        ],
        scratch_types=[
            pltpu.VMEM((_CH,), jnp.int32),
            pltpu.VMEM((_CH,), f32),
            pltpu.VMEM((_CH,), f32),
            pltpu.VMEM((_CH,), f32),
            pltpu.VMEM((_CH,), f32),
            pltpu.VMEM((_SORT_N * 16,), jnp.int32),
            pltpu.VMEM((_SORT_N * 16,), jnp.int32),
            pltpu.VMEM((_CH,), jnp.int32),
            pltpu.VMEM((_CH,), jnp.int32),
            pltpu.VMEM((_CH,), f32),
            pltpu.VMEM((_CH,), f32),
            pltpu.VMEM((_CH,), f32),
            pltpu.VMEM((_CH,), f32),
        ],
    )(_sort_body)
    us, gs, sy1a, sx1a, sy2a, sx2a = sort_k(ukey, y1f, x1f, y2f, x2f)

    scan_k = functools.partial(
        pl.kernel, mesh=mesh, compiler_params=sc_params,
        out_type=jax.ShapeDtypeStruct((_MAX_OUT * 16,), f32),
        scratch_types=[
            pltpu.VMEM((_P,), jnp.int32),
            pltpu.VMEM((_P,), jnp.int32),
            pltpu.VMEM((_P,), f32),
            pltpu.VMEM((_P,), f32),
            pltpu.VMEM((_P,), f32),
            pltpu.VMEM((_P,), f32),
            pltpu.VMEM((512,), jnp.int32),
            pltpu.VMEM((512,), jnp.int32),
            pltpu.VMEM((512,), jnp.int32),
            pltpu.VMEM((32,), jnp.int32),
            pltpu.VMEM((32,), jnp.int32),
            pltpu.VMEM((112,), f32),
            pltpu.VMEM((112,), f32),
            pltpu.VMEM((112,), f32),
            pltpu.VMEM((112,), f32),
            pltpu.VMEM((112,), f32),
            pltpu.VMEM((_MAX_OUT * 16,), f32),
        ],
    )(_scan_body)
    det = scan_k(us, gs, sy1a, sx1a, sy2a, sx2a)

    return det.reshape(_MAX_OUT, 16)[:, :5].reshape(B, _MAX_OUT, 5)


# speedup vs baseline: 1.0449x; 1.0449x over previous
import functools

import jax
import jax.numpy as jnp
from jax import lax
from jax.experimental import pallas as pl
from jax.experimental.pallas import tpu as pltpu
from jax.experimental.pallas import tpu_sc as plsc

_BBOX_STD = (0.1, 0.1, 0.2, 0.2)
_MAX_OUT = 100
_NMS_THR = 0.3
_CONF = 0.15
_NEG = -1e30

_ROWS = 160
_LANES = 128
_P = _ROWS * _LANES
_NT = 32
_CH = _P // _NT
_NC = 16
_D = _CH // _NC
_SORT_N = 64
_BIGI = jnp.int32(2**30)


def _prep_body(boxes_ref, deltas_ref, probs_ref, meta_ref,
               ukey_ref, y1_ref, x1_ref, y2_ref, x2_ref):
    h = meta_ref[0, 4]
    w = meta_ref[0, 5]
    wy1 = (meta_ref[0, 7] - 0.0) / (h - 1.0)
    wx1 = (meta_ref[0, 8] - 0.0) / (w - 1.0)
    wy2 = (meta_ref[0, 9] - 1.0) / (h - 1.0)
    wx2 = (meta_ref[0, 10] - 1.0) / (w - 1.0)

    ry1 = boxes_ref[0]
    rx1 = boxes_ref[1]
    ry2 = boxes_ref[2]
    rx2 = boxes_ref[3]
    dy = deltas_ref[0] * _BBOX_STD[0]
    dx = deltas_ref[1] * _BBOX_STD[1]
    dh = deltas_ref[2] * _BBOX_STD[2]
    dw = deltas_ref[3] * _BBOX_STD[3]

    height = ry2 - ry1
    width = rx2 - rx1
    cy = ry1 + 0.5 * height + dy * height
    cx = rx1 + 0.5 * width + dx * width
    height = height * jnp.exp(dh)
    width = width * jnp.exp(dw)
    y1 = cy - 0.5 * height
    x1 = cx - 0.5 * width
    y2 = y1 + height
    x2 = x1 + width

    y1_ref[...] = jnp.clip(y1, wy1, wy2)
    x1_ref[...] = jnp.clip(x1, wx1, wx2)
    y2_ref[...] = jnp.clip(y2, wy1, wy2)
    x2_ref[...] = jnp.clip(x2, wx1, wx2)

    probs = probs_ref[...]
    bits = lax.bitcast_convert_type(probs, jnp.int32)
    ukey_ref[...] = jnp.where(probs >= _CONF, bits, jnp.int32(-1))


def _lane():
    return lax.broadcasted_iota(jnp.int32, (16,), 0)


def _sort_body(ukey_hbm, y1_hbm, x1_hbm, y2_hbm, x2_hbm,
               us_hbm, gs_hbm, sy1_hbm, sx1_hbm, sy2_hbm, sx2_hbm,
               uloc, y1l, x1l, y2l, x2l, key, gid,
               e_us, e_gs, e_y1, e_x1, e_y2, e_x2):
    tid = lax.axis_index("s") * 2 + lax.axis_index("c")
    base = tid * _CH
    LANE = _lane()

    pltpu.sync_copy(ukey_hbm.at[pl.ds(base, _CH)], uloc)
    pltpu.sync_copy(y1_hbm.at[pl.ds(base, _CH)], y1l)
    pltpu.sync_copy(x1_hbm.at[pl.ds(base, _CH)], x1l)
    pltpu.sync_copy(y2_hbm.at[pl.ds(base, _CH)], y2l)
    pltpu.sync_copy(x2_hbm.at[pl.ds(base, _CH)], x2l)

    for r in range(_SORT_N):
        if r < _D:
            kv = uloc[pl.ds(r * 16, 16)]
            gv = base + r * 16 + LANE
        else:
            kv = jnp.full((16,), -1, jnp.int32)
            gv = jnp.full((16,), _P, jnp.int32)
        key[pl.ds(r * 16, 16)] = kv
        gid[pl.ds(r * 16, 16)] = gv

    k = 2
    while k <= _SORT_N:
        j = k // 2
        while j >= 1:
            sh = j.bit_length() - 1

            def ce(i, _, j=j, k=k, sh=sh):
                a = ((i >> sh) << (sh + 1)) | (i & (j - 1))
                b = a | j
                ka = key[pl.ds(a * 16, 16)]
                kb = key[pl.ds(b * 16, 16)]
                ga = gid[pl.ds(a * 16, 16)]
                gb = gid[pl.ds(b * 16, 16)]
                agtb = (ka > kb) | ((ka == kb) & (ga < gb))
                kg = jnp.where(agtb, ka, kb)
                kl = jnp.where(agtb, kb, ka)
                gg = jnp.where(agtb, ga, gb)
                gl = jnp.where(agtb, gb, ga)
                desc = (a & k) == 0
                key[pl.ds(a * 16, 16)] = jnp.where(desc, kg, kl)
                key[pl.ds(b * 16, 16)] = jnp.where(desc, kl, kg)
                gid[pl.ds(a * 16, 16)] = jnp.where(desc, gg, gl)
                gid[pl.ds(b * 16, 16)] = jnp.where(desc, gl, gg)
                return 0

            lax.fori_loop(0, _SORT_N // 2, ce, 0)
            j //= 2
        k *= 2

    for c in range(_NC):
        for ch in range((_D + 15) // 16):
            dvec = ch * 16 + LANE
            msk = dvec < _D
            src = jnp.minimum(dvec, _SORT_N - 1) * 16 + c
            kv = plsc.load_gather(key, [src])
            gv = plsc.load_gather(gid, [src])
            lidx = jnp.clip(gv - base, 0, _CH - 1)
            dst = jnp.minimum(c * _D + dvec, _CH - 1)
            plsc.store_scatter(e_us, [dst], kv, mask=msk)
            plsc.store_scatter(e_gs, [dst], gv, mask=msk)
            plsc.store_scatter(e_y1, [dst], plsc.load_gather(y1l, [lidx]),
                               mask=msk)
            plsc.store_scatter(e_x1, [dst], plsc.load_gather(x1l, [lidx]),
                               mask=msk)
            plsc.store_scatter(e_y2, [dst], plsc.load_gather(y2l, [lidx]),
                               mask=msk)
            plsc.store_scatter(e_x2, [dst], plsc.load_gather(x2l, [lidx]),
                               mask=msk)

    pltpu.sync_copy(e_us, us_hbm.at[pl.ds(base, _CH)])
    pltpu.sync_copy(e_gs, gs_hbm.at[pl.ds(base, _CH)])
    pltpu.sync_copy(e_y1, sy1_hbm.at[pl.ds(base, _CH)])
    pltpu.sync_copy(e_x1, sx1_hbm.at[pl.ds(base, _CH)])
    pltpu.sync_copy(e_y2, sy2_hbm.at[pl.ds(base, _CH)])
    pltpu.sync_copy(e_x2, sx2_hbm.at[pl.ds(base, _CH)])


def _splat_i(x):
    return jnp.zeros((16,), jnp.int32) + x


def _splat_f(x):
    return jnp.zeros((16,), jnp.float32) + x


def _scan_body(us_hbm, gs_hbm, sy1_hbm, sx1_hbm, sy2_hbm, sx2_hbm,
               det_hbm,
               su, sg, sy1, sx1, sy2, sx2,
               head_u, head_g, head_d, grp_u, grp_g,
               v_y1, v_x1, v_y2, v_x2, v_ar, outb):
    wid = lax.axis_index("s") * 2 + lax.axis_index("c")
    LANE = _lane()
    NS = _NT * _NC

    @pl.when(wid == 0)
    def _():
        pltpu.sync_copy(us_hbm, su)
        pltpu.sync_copy(gs_hbm, sg)
        pltpu.sync_copy(sy1_hbm, sy1)
        pltpu.sync_copy(sx1_hbm, sx1)
        pltpu.sync_copy(sy2_hbm, sy2)
        pltpu.sync_copy(sx2_hbm, sx2)

        zf = jnp.zeros((16,), jnp.float32)
        for jj in range(7):
            v_y1[pl.ds(jj * 16, 16)] = zf
            v_x1[pl.ds(jj * 16, 16)] = zf
            v_y2[pl.ds(jj * 16, 16)] = zf
            v_x2[pl.ds(jj * 16, 16)] = zf
            v_ar[pl.ds(jj * 16, 16)] = zf
        for r in range(_MAX_OUT):
            outb[pl.ds(r * 16, 16)] = zf

        def init_heads(i, _):
            svec = i * 16 + LANE
            posv = svec * _D
            hu = plsc.load_gather(su, [posv])
            hg = plsc.load_gather(sg, [posv])
            plsc.store_scatter(head_u, [svec], hu)
            plsc.store_scatter(head_g, [svec], hg)
            plsc.store_scatter(head_d, [svec], jnp.zeros((16,), jnp.int32))
            gm = jnp.max(hu)
            gg = jnp.min(jnp.where(hu == gm, hg, _BIGI))
            m0 = LANE == 0
            plsc.store_scatter(grp_u, [_splat_i(i)], _splat_i(gm), mask=m0)
            plsc.store_scatter(grp_g, [_splat_i(i)], _splat_i(gg), mask=m0)
            return 0

        lax.fori_loop(0, NS // 16, init_heads, 0)

        def body(carry):
            count, _go = carry
            g0u = grp_u[pl.ds(0, 16)]
            g1u = grp_u[pl.ds(16, 16)]
            g0g = grp_g[pl.ds(0, 16)]
            g1g = grp_g[pl.ds(16, 16)]
            mu = jnp.maximum(jnp.max(g0u), jnp.max(g1u))
            valid = mu >= 0
            mg = jnp.minimum(
                jnp.min(jnp.where(g0u == mu, g0g, _BIGI)),
                jnp.min(jnp.where(g1u == mu, g1g, _BIGI)))
            h0 = (g0u == mu) & (g0g == mg)
            h1 = (g1u == mu) & (g1g == mg)
            g = jnp.minimum(
                jnp.min(jnp.where(h0, LANE, jnp.int32(99))),
                jnp.min(jnp.where(h1, LANE + 16, jnp.int32(99))))
            g = jnp.minimum(g, jnp.int32(31))
            hvec = g * 16 + LANE
            hu16 = head_u[pl.ds(g * 16, 16)]
            hg16 = head_g[pl.ds(g * 16, 16)]
            lsel = jnp.min(jnp.where((hu16 == mu) & (hg16 == mg), LANE,
                                     jnp.int32(15)))
            s = jnp.minimum(g * 16 + lsel, jnp.int32(NS - 1))
            d = jnp.max(plsc.load_gather(head_d, [_splat_i(s)]))
            pos = jnp.minimum(s * _D + d, jnp.int32(_P - 1))
            posv = _splat_i(pos)
            y1b = jnp.max(plsc.load_gather(sy1, [posv]))
            x1b = jnp.max(plsc.load_gather(sx1, [posv]))
            y2b = jnp.max(plsc.load_gather(sy2, [posv]))
            x2b = jnp.max(plsc.load_gather(sx2, [posv]))
            area_b = (y2b - y1b) * (x2b - x1b)

            def iou_blk(jblk, acc):
                ya = v_y1[pl.ds(jblk * 16, 16)]
                xa = v_x1[pl.ds(jblk * 16, 16)]
                yb2 = v_y2[pl.ds(jblk * 16, 16)]
                xb2 = v_x2[pl.ds(jblk * 16, 16)]
                ar = v_ar[pl.ds(jblk * 16, 16)]
                yy1 = jnp.maximum(ya, y1b)
                xx1 = jnp.maximum(xa, x1b)
                yy2 = jnp.minimum(yb2, y2b)
                xx2 = jnp.minimum(xb2, x2b)
                inter = (jnp.maximum(yy2 - yy1, 0.0)
                         * jnp.maximum(xx2 - xx1, 0.0))
                denom = ar + area_b - inter + 1e-9
                return jnp.maximum(acc, inter - _NMS_THR * denom)

            acc = lax.fori_loop(0, 7, iou_blk, jnp.full((16,), -1.0,
                                                        jnp.float32))
            sup = jnp.max(acc) > 0.0
            accept = valid & jnp.logical_not(sup)

            m0a = (LANE == 0) & accept
            cidx = _splat_i(count)
            plsc.store_scatter(v_y1, [cidx], _splat_f(y1b), mask=m0a)
            plsc.store_scatter(v_x1, [cidx], _splat_f(x1b), mask=m0a)
            plsc.store_scatter(v_y2, [cidx], _splat_f(y2b), mask=m0a)
            plsc.store_scatter(v_x2, [cidx], _splat_f(x2b), mask=m0a)
            plsc.store_scatter(v_ar, [cidx], _splat_f(area_b), mask=m0a)

            score = jnp.max(plsc.bitcast(_splat_i(mu), jnp.float32))
            row = jnp.where(
                LANE == 0, y1b,
                jnp.where(LANE == 1, x1b,
                          jnp.where(LANE == 2, y2b,
                                    jnp.where(LANE == 3, x2b,
                                              jnp.where(LANE == 4, score,
                                                        0.0)))))
            outm = (jnp.zeros((16,), jnp.bool_) | accept)
            plsc.store_scatter(outb, [count * 16 + LANE], row, mask=outm)
            count2 = count + jnp.where(accept, 1, 0).astype(jnp.int32)

            nd = d + 1
            ex = nd >= _D
            npos = _splat_i(jnp.minimum(pos + 1, jnp.int32(_P - 1)))
            nu = jnp.where(ex, jnp.int32(-1),
                           jnp.max(plsc.load_gather(su, [npos])))
            ng = jnp.where(ex, _BIGI, jnp.max(plsc.load_gather(sg, [npos])))
            m0v = (LANE == 0) & valid
            sv_ = _splat_i(s)
            plsc.store_scatter(head_u, [sv_], _splat_i(nu), mask=m0v)
            plsc.store_scatter(head_g, [sv_], _splat_i(ng), mask=m0v)
            plsc.store_scatter(head_d, [sv_], _splat_i(nd), mask=m0v)
            hu2 = head_u[pl.ds(g * 16, 16)]
            hg2 = head_g[pl.ds(g * 16, 16)]
            gm2 = jnp.max(hu2)
            gg2 = jnp.min(jnp.where(hu2 == gm2, hg2, _BIGI))
            gidxv = _splat_i(g)
            plsc.store_scatter(grp_u, [gidxv], _splat_i(gm2), mask=m0v)
            plsc.store_scatter(grp_g, [gidxv], _splat_i(gg2), mask=m0v)

            go2 = valid & (count2 < _MAX_OUT)
            return (count2, go2)

        lax.while_loop(lambda c: c[1], body,
                       (jnp.int32(0), jnp.bool_(True)))

        pltpu.sync_copy(outb, det_hbm)


@functools.partial(jax.jit, static_argnames=())
def kernel(rois, bbox_scores, macacnn_bbox, image_meta):
    B, N = rois.shape[0], rois.shape[1]
    pad = _P - N

    def prep(x, fill):
        xt = jnp.transpose(x, (1, 0))
        xt = jnp.pad(xt, ((0, 0), (0, pad)), constant_values=fill)
        return xt.reshape(x.shape[1], _ROWS, _LANES)

    boxes_in = prep(rois[0], 0.0)
    deltas_in = prep(macacnn_bbox[0], 0.0)
    probs_in = prep(bbox_scores[0], -1.0)[0]

    f32 = jnp.float32
    ukey2d, y1p, x1p, y2p, x2p = pl.pallas_call(
        _prep_body,
        out_shape=[
            jax.ShapeDtypeStruct((_ROWS, _LANES), jnp.int32),
            jax.ShapeDtypeStruct((_ROWS, _LANES), f32),
            jax.ShapeDtypeStruct((_ROWS, _LANES), f32),
            jax.ShapeDtypeStruct((_ROWS, _LANES), f32),
            jax.ShapeDtypeStruct((_ROWS, _LANES), f32),
        ],
        in_specs=[
            pl.BlockSpec(memory_space=pltpu.VMEM),
            pl.BlockSpec(memory_space=pltpu.VMEM),
            pl.BlockSpec(memory_space=pltpu.VMEM),
            pl.BlockSpec(memory_space=pltpu.SMEM),
        ],
        out_specs=[pl.BlockSpec(memory_space=pltpu.VMEM)] * 5,
    )(boxes_in, deltas_in, probs_in, image_meta)

    ukey = ukey2d.reshape(_P)
    y1f = y1p.reshape(_P)
    x1f = x1p.reshape(_P)
    y2f = y2p.reshape(_P)
    x2f = x2p.reshape(_P)

    mesh = plsc.VectorSubcoreMesh(core_axis_name="c", subcore_axis_name="s")

    sc_params = pltpu.CompilerParams(needs_layout_passes=False)

    sort_k = functools.partial(
        pl.kernel, mesh=mesh, compiler_params=sc_params,
        out_type=[
            jax.ShapeDtypeStruct((_P,), jnp.int32),
            jax.ShapeDtypeStruct((_P,), jnp.int32),
            jax.ShapeDtypeStruct((_P,), f32),
            jax.ShapeDtypeStruct((_P,), f32),
            jax.ShapeDtypeStruct((_P,), f32),
            jax.ShapeDtypeStruct((_P,), f32),
        ],
        scratch_types=[
            pltpu.VMEM((_CH,), jnp.int32),
            pltpu.VMEM((_CH,), f32),
            pltpu.VMEM((_CH,), f32),
            pltpu.VMEM((_CH,), f32),
            pltpu.VMEM((_CH,), f32),
            pltpu.VMEM((_SORT_N * 16,), jnp.int32),
            pltpu.VMEM((_SORT_N * 16,), jnp.int32),
            pltpu.VMEM((_CH,), jnp.int32),
            pltpu.VMEM((_CH,), jnp.int32),
            pltpu.VMEM((_CH,), f32),
            pltpu.VMEM((_CH,), f32),
            pltpu.VMEM((_CH,), f32),
            pltpu.VMEM((_CH,), f32),
        ],
    )(_sort_body)
    us, gs, sy1a, sx1a, sy2a, sx2a = sort_k(ukey, y1f, x1f, y2f, x2f)

    scan_k = functools.partial(
        pl.kernel, mesh=mesh, compiler_params=sc_params,
        out_type=jax.ShapeDtypeStruct((_MAX_OUT * 16,), f32),
        scratch_types=[
            pltpu.VMEM((_P,), jnp.int32),
            pltpu.VMEM((_P,), jnp.int32),
            pltpu.VMEM((_P,), f32),
            pltpu.VMEM((_P,), f32),
            pltpu.VMEM((_P,), f32),
            pltpu.VMEM((_P,), f32),
            pltpu.VMEM((512,), jnp.int32),
            pltpu.VMEM((512,), jnp.int32),
            pltpu.VMEM((512,), jnp.int32),
            pltpu.VMEM((32,), jnp.int32),
            pltpu.VMEM((32,), jnp.int32),
            pltpu.VMEM((112,), f32),
            pltpu.VMEM((112,), f32),
            pltpu.VMEM((112,), f32),
            pltpu.VMEM((112,), f32),
            pltpu.VMEM((112,), f32),
            pltpu.VMEM((_MAX_OUT * 16,), f32),
        ],
    )(_scan_body)
    det = scan_k(us, gs, sy1a, sx1a, sy2a, sx2a)

    return det.reshape(_MAX_OUT, 16)[:, :5].reshape(B, _MAX_OUT, 5)


# speedup vs baseline: 1.0744x; 1.0282x over previous
import functools

import jax
import jax.numpy as jnp
from jax import lax
from jax.experimental import pallas as pl
from jax.experimental.pallas import tpu as pltpu
from jax.experimental.pallas import tpu_sc as plsc

_BBOX_STD = (0.1, 0.1, 0.2, 0.2)
_MAX_OUT = 100
_NMS_THR = 0.3
_CONF = 0.15
_NEG = -1e30

_ROWS = 160
_LANES = 128
_P = _ROWS * _LANES
_NT = 32
_CH = _P // _NT
_NC = 16
_D = _CH // _NC
_SORT_N = 64
_BIGI = jnp.int32(2**30)


def _prep_body(boxes_ref, deltas_ref, probs_ref, meta_ref,
               ukey_ref, y1_ref, x1_ref, y2_ref, x2_ref):
    h = meta_ref[0, 4]
    w = meta_ref[0, 5]
    wy1 = (meta_ref[0, 7] - 0.0) / (h - 1.0)
    wx1 = (meta_ref[0, 8] - 0.0) / (w - 1.0)
    wy2 = (meta_ref[0, 9] - 1.0) / (h - 1.0)
    wx2 = (meta_ref[0, 10] - 1.0) / (w - 1.0)

    ry1 = boxes_ref[0]
    rx1 = boxes_ref[1]
    ry2 = boxes_ref[2]
    rx2 = boxes_ref[3]
    dy = deltas_ref[0] * _BBOX_STD[0]
    dx = deltas_ref[1] * _BBOX_STD[1]
    dh = deltas_ref[2] * _BBOX_STD[2]
    dw = deltas_ref[3] * _BBOX_STD[3]

    height = ry2 - ry1
    width = rx2 - rx1
    cy = ry1 + 0.5 * height + dy * height
    cx = rx1 + 0.5 * width + dx * width
    height = height * jnp.exp(dh)
    width = width * jnp.exp(dw)
    y1 = cy - 0.5 * height
    x1 = cx - 0.5 * width
    y2 = y1 + height
    x2 = x1 + width

    y1_ref[...] = jnp.clip(y1, wy1, wy2)
    x1_ref[...] = jnp.clip(x1, wx1, wx2)
    y2_ref[...] = jnp.clip(y2, wy1, wy2)
    x2_ref[...] = jnp.clip(x2, wx1, wx2)

    probs = probs_ref[...]
    bits = lax.bitcast_convert_type(probs, jnp.int32)
    ukey_ref[...] = jnp.where(probs >= _CONF, bits, jnp.int32(-1))


def _lane():
    return lax.broadcasted_iota(jnp.int32, (16,), 0)


def _sort_body(ukey_hbm, y1_hbm, x1_hbm, y2_hbm, x2_hbm,
               us_hbm, gs_hbm, sy1_hbm, sx1_hbm, sy2_hbm, sx2_hbm,
               uloc, y1l, x1l, y2l, x2l, key, gid,
               e_us, e_gs, e_y1, e_x1, e_y2, e_x2):
    tid = lax.axis_index("s") * 2 + lax.axis_index("c")
    base = tid * _CH
    LANE = _lane()

    pltpu.sync_copy(ukey_hbm.at[pl.ds(base, _CH)], uloc)
    pltpu.sync_copy(y1_hbm.at[pl.ds(base, _CH)], y1l)
    pltpu.sync_copy(x1_hbm.at[pl.ds(base, _CH)], x1l)
    pltpu.sync_copy(y2_hbm.at[pl.ds(base, _CH)], y2l)
    pltpu.sync_copy(x2_hbm.at[pl.ds(base, _CH)], x2l)

    for r in range(_SORT_N):
        if r < _D:
            kv = uloc[pl.ds(r * 16, 16)]
            gv = base + r * 16 + LANE
        else:
            kv = jnp.full((16,), -1, jnp.int32)
            gv = jnp.full((16,), _P, jnp.int32)
        key[pl.ds(r * 16, 16)] = kv
        gid[pl.ds(r * 16, 16)] = gv

    k = 2
    while k <= _SORT_N:
        j = k // 2
        while j >= 1:
            sh = j.bit_length() - 1

            def ce(i, _, j=j, k=k, sh=sh):
                a = ((i >> sh) << (sh + 1)) | (i & (j - 1))
                b = a | j
                ka = key[pl.ds(a * 16, 16)]
                kb = key[pl.ds(b * 16, 16)]
                ga = gid[pl.ds(a * 16, 16)]
                gb = gid[pl.ds(b * 16, 16)]
                agtb = (ka > kb) | ((ka == kb) & (ga < gb))
                kg = jnp.where(agtb, ka, kb)
                kl = jnp.where(agtb, kb, ka)
                gg = jnp.where(agtb, ga, gb)
                gl = jnp.where(agtb, gb, ga)
                desc = (a & k) == 0
                key[pl.ds(a * 16, 16)] = jnp.where(desc, kg, kl)
                key[pl.ds(b * 16, 16)] = jnp.where(desc, kl, kg)
                gid[pl.ds(a * 16, 16)] = jnp.where(desc, gg, gl)
                gid[pl.ds(b * 16, 16)] = jnp.where(desc, gl, gg)
                return 0

            lax.fori_loop(0, _SORT_N // 2, ce, 0)
            j //= 2
        k *= 2

    for c in range(_NC):
        for ch in range((_D + 15) // 16):
            dvec = ch * 16 + LANE
            msk = dvec < _D
            src = jnp.minimum(dvec, _SORT_N - 1) * 16 + c
            kv = plsc.load_gather(key, [src])
            gv = plsc.load_gather(gid, [src])
            lidx = jnp.clip(gv - base, 0, _CH - 1)
            dst = jnp.minimum(c * _D + dvec, _CH - 1)
            plsc.store_scatter(e_us, [dst], kv, mask=msk)
            plsc.store_scatter(e_gs, [dst], gv, mask=msk)
            plsc.store_scatter(e_y1, [dst], plsc.load_gather(y1l, [lidx]),
                               mask=msk)
            plsc.store_scatter(e_x1, [dst], plsc.load_gather(x1l, [lidx]),
                               mask=msk)
            plsc.store_scatter(e_y2, [dst], plsc.load_gather(y2l, [lidx]),
                               mask=msk)
            plsc.store_scatter(e_x2, [dst], plsc.load_gather(x2l, [lidx]),
                               mask=msk)

    pltpu.sync_copy(e_us, us_hbm.at[pl.ds(base, _CH)])
    pltpu.sync_copy(e_gs, gs_hbm.at[pl.ds(base, _CH)])
    pltpu.sync_copy(e_y1, sy1_hbm.at[pl.ds(base, _CH)])
    pltpu.sync_copy(e_x1, sx1_hbm.at[pl.ds(base, _CH)])
    pltpu.sync_copy(e_y2, sy2_hbm.at[pl.ds(base, _CH)])
    pltpu.sync_copy(e_x2, sx2_hbm.at[pl.ds(base, _CH)])


def _splat_i(x):
    return jnp.zeros((16,), jnp.int32) + x


def _splat_f(x):
    return jnp.zeros((16,), jnp.float32) + x


def _scan_body(us_hbm, gs_hbm, sy1_hbm, sx1_hbm, sy2_hbm, sx2_hbm,
               det_hbm,
               su, sg, sy1, sx1, sy2, sx2,
               head_u, head_g, head_d, grp_u, grp_g,
               v_y1, v_x1, v_y2, v_x2, v_ar, outb):
    wid = lax.axis_index("s") * 2 + lax.axis_index("c")
    LANE = _lane()
    NS = _NT * _NC

    @pl.when(wid == 0)
    def _():
        pltpu.sync_copy(us_hbm, su)
        pltpu.sync_copy(gs_hbm, sg)
        pltpu.sync_copy(sy1_hbm, sy1)
        pltpu.sync_copy(sx1_hbm, sx1)
        pltpu.sync_copy(sy2_hbm, sy2)
        pltpu.sync_copy(sx2_hbm, sx2)

        zf = jnp.zeros((16,), jnp.float32)
        for jj in range(7):
            v_y1[pl.ds(jj * 16, 16)] = zf
            v_x1[pl.ds(jj * 16, 16)] = zf
            v_y2[pl.ds(jj * 16, 16)] = zf
            v_x2[pl.ds(jj * 16, 16)] = zf
            v_ar[pl.ds(jj * 16, 16)] = zf
        for r in range(_MAX_OUT):
            outb[pl.ds(r * 16, 16)] = zf

        def init_heads(i, _):
            svec = i * 16 + LANE
            posv = svec * _D
            hu = plsc.load_gather(su, [posv])
            hg = plsc.load_gather(sg, [posv])
            plsc.store_scatter(head_u, [svec], hu)
            plsc.store_scatter(head_g, [svec], hg)
            plsc.store_scatter(head_d, [svec], posv)
            gm = jnp.max(hu)
            gg = jnp.min(jnp.where(hu == gm, hg, _BIGI))
            m0 = LANE == 0
            plsc.store_scatter(grp_u, [_splat_i(i)], _splat_i(gm), mask=m0)
            plsc.store_scatter(grp_g, [_splat_i(i)], _splat_i(gg), mask=m0)
            return 0

        lax.fori_loop(0, NS // 16, init_heads, 0)

        def body(carry):
            count, _go = carry
            g0u = grp_u[pl.ds(0, 16)]
            g1u = grp_u[pl.ds(16, 16)]
            g0g = grp_g[pl.ds(0, 16)]
            g1g = grp_g[pl.ds(16, 16)]
            mu = jnp.max(jnp.maximum(g0u, g1u))
            valid = mu >= 0
            mg = jnp.min(jnp.minimum(
                jnp.where(g0u == mu, g0g, _BIGI),
                jnp.where(g1u == mu, g1g, _BIGI)))
            h0 = (g0u == mu) & (g0g == mg)
            h1 = (g1u == mu) & (g1g == mg)
            g = jnp.min(jnp.minimum(
                jnp.where(h0, LANE, jnp.int32(99)),
                jnp.where(h1, LANE + 16, jnp.int32(99))))
            g = jnp.minimum(g, jnp.int32(31))
            hu16 = head_u[pl.ds(g * 16, 16)]
            hg16 = head_g[pl.ds(g * 16, 16)]
            lsel = jnp.min(jnp.where((hu16 == mu) & (hg16 == mg), LANE,
                                     jnp.int32(15)))
            s = jnp.minimum(g * 16 + lsel, jnp.int32(NS - 1))
            sv_ = _splat_i(s)
            posv = jnp.minimum(plsc.load_gather(head_d, [sv_]),
                               jnp.int32(_P - 1))
            y1v = plsc.load_gather(sy1, [posv])
            x1v = plsc.load_gather(sx1, [posv])
            y2v = plsc.load_gather(sy2, [posv])
            x2v = plsc.load_gather(sx2, [posv])
            arv = (y2v - y1v) * (x2v - x1v)

            def iou_blk(jblk, acc):
                ya = v_y1[pl.ds(jblk * 16, 16)]
                xa = v_x1[pl.ds(jblk * 16, 16)]
                yb2 = v_y2[pl.ds(jblk * 16, 16)]
                xb2 = v_x2[pl.ds(jblk * 16, 16)]
                ar = v_ar[pl.ds(jblk * 16, 16)]
                yy1 = jnp.maximum(ya, y1v)
                xx1 = jnp.maximum(xa, x1v)
                yy2 = jnp.minimum(yb2, y2v)
                xx2 = jnp.minimum(xb2, x2v)
                inter = (jnp.maximum(yy2 - yy1, 0.0)
                         * jnp.maximum(xx2 - xx1, 0.0))
                denom = ar + arv - inter + 1e-9
                return jnp.maximum(acc, inter - _NMS_THR * denom)

            acc = lax.fori_loop(0, 7, iou_blk, jnp.full((16,), -1.0,
                                                        jnp.float32))
            sup = jnp.max(acc) > 0.0
            accept = valid & jnp.logical_not(sup)

            m0a = (LANE == 0) & accept
            cidx = _splat_i(count)
            plsc.store_scatter(v_y1, [cidx], y1v, mask=m0a)
            plsc.store_scatter(v_x1, [cidx], x1v, mask=m0a)
            plsc.store_scatter(v_y2, [cidx], y2v, mask=m0a)
            plsc.store_scatter(v_x2, [cidx], x2v, mask=m0a)
            plsc.store_scatter(v_ar, [cidx], arv, mask=m0a)

            scorev = plsc.bitcast(_splat_i(mu), jnp.float32)
            zfv = jnp.zeros((16,), jnp.float32)
            row = jnp.where(
                LANE == 0, y1v,
                jnp.where(LANE == 1, x1v,
                          jnp.where(LANE == 2, y2v,
                                    jnp.where(LANE == 3, x2v,
                                              jnp.where(LANE == 4, scorev,
                                                        zfv)))))
            outm = (jnp.zeros((16,), jnp.bool_) | accept)
            plsc.store_scatter(outb, [count * 16 + LANE], row, mask=outm)
            count2 = count + jnp.where(accept, 1, 0).astype(jnp.int32)

            nextv = posv + 1
            ex_v = (nextv % _D) == 0
            safe_next = jnp.minimum(nextv, jnp.int32(_P - 1))
            nu = jnp.where(ex_v, jnp.int32(-1),
                           plsc.load_gather(su, [safe_next]))
            ng = jnp.where(ex_v, _BIGI, plsc.load_gather(sg, [safe_next]))
            m0v = (LANE == 0) & valid
            plsc.store_scatter(head_u, [sv_], nu, mask=m0v)
            plsc.store_scatter(head_g, [sv_], ng, mask=m0v)
            plsc.store_scatter(head_d, [sv_], nextv, mask=m0v)
            hu2 = head_u[pl.ds(g * 16, 16)]
            hg2 = head_g[pl.ds(g * 16, 16)]
            gm2 = jnp.max(hu2)
            gg2 = jnp.min(jnp.where(hu2 == gm2, hg2, _BIGI))
            gidxv = _splat_i(g)
            plsc.store_scatter(grp_u, [gidxv], _splat_i(gm2), mask=m0v)
            plsc.store_scatter(grp_g, [gidxv], _splat_i(gg2), mask=m0v)

            go2 = valid & (count2 < _MAX_OUT)
            return (count2, go2)

        lax.while_loop(lambda c: c[1], body,
                       (jnp.int32(0), jnp.bool_(True)))

        pltpu.sync_copy(outb, det_hbm)


@functools.partial(jax.jit, static_argnames=())
def kernel(rois, bbox_scores, macacnn_bbox, image_meta):
    B, N = rois.shape[0], rois.shape[1]
    pad = _P - N

    def prep(x, fill):
        xt = jnp.transpose(x, (1, 0))
        xt = jnp.pad(xt, ((0, 0), (0, pad)), constant_values=fill)
        return xt.reshape(x.shape[1], _ROWS, _LANES)

    boxes_in = prep(rois[0], 0.0)
    deltas_in = prep(macacnn_bbox[0], 0.0)
    probs_in = prep(bbox_scores[0], -1.0)[0]

    f32 = jnp.float32
    ukey2d, y1p, x1p, y2p, x2p = pl.pallas_call(
        _prep_body,
        out_shape=[
            jax.ShapeDtypeStruct((_ROWS, _LANES), jnp.int32),
            jax.ShapeDtypeStruct((_ROWS, _LANES), f32),
            jax.ShapeDtypeStruct((_ROWS, _LANES), f32),
            jax.ShapeDtypeStruct((_ROWS, _LANES), f32),
            jax.ShapeDtypeStruct((_ROWS, _LANES), f32),
        ],
        in_specs=[
            pl.BlockSpec(memory_space=pltpu.VMEM),
            pl.BlockSpec(memory_space=pltpu.VMEM),
            pl.BlockSpec(memory_space=pltpu.VMEM),
            pl.BlockSpec(memory_space=pltpu.SMEM),
        ],
        out_specs=[pl.BlockSpec(memory_space=pltpu.VMEM)] * 5,
    )(boxes_in, deltas_in, probs_in, image_meta)

    ukey = ukey2d.reshape(_P)
    y1f = y1p.reshape(_P)
    x1f = x1p.reshape(_P)
    y2f = y2p.reshape(_P)
    x2f = x2p.reshape(_P)

    mesh = plsc.VectorSubcoreMesh(core_axis_name="c", subcore_axis_name="s")

    sc_params = pltpu.CompilerParams(needs_layout_passes=False)

    sort_k = functools.partial(
        pl.kernel, mesh=mesh, compiler_params=sc_params,
        out_type=[
            jax.ShapeDtypeStruct((_P,), jnp.int32),
            jax.ShapeDtypeStruct((_P,), jnp.int32),
            jax.ShapeDtypeStruct((_P,), f32),
            jax.ShapeDtypeStruct((_P,), f32),
            jax.ShapeDtypeStruct((_P,), f32),
            jax.ShapeDtypeStruct((_P,), f32),
        ],
        scratch_types=[
            pltpu.VMEM((_CH,), jnp.int32),
            pltpu.VMEM((_CH,), f32),
            pltpu.VMEM((_CH,), f32),
            pltpu.VMEM((_CH,), f32),
            pltpu.VMEM((_CH,), f32),
            pltpu.VMEM((_SORT_N * 16,), jnp.int32),
            pltpu.VMEM((_SORT_N * 16,), jnp.int32),
            pltpu.VMEM((_CH,), jnp.int32),
            pltpu.VMEM((_CH,), jnp.int32),
            pltpu.VMEM((_CH,), f32),
            pltpu.VMEM((_CH,), f32),
            pltpu.VMEM((_CH,), f32),
            pltpu.VMEM((_CH,), f32),
        ],
    )(_sort_body)
    us, gs, sy1a, sx1a, sy2a, sx2a = sort_k(ukey, y1f, x1f, y2f, x2f)

    scan_k = functools.partial(
        pl.kernel, mesh=mesh, compiler_params=sc_params,
        out_type=jax.ShapeDtypeStruct((_MAX_OUT * 16,), f32),
        scratch_types=[
            pltpu.VMEM((_P,), jnp.int32),
            pltpu.VMEM((_P,), jnp.int32),
            pltpu.VMEM((_P,), f32),
            pltpu.VMEM((_P,), f32),
            pltpu.VMEM((_P,), f32),
            pltpu.VMEM((_P,), f32),
            pltpu.VMEM((512,), jnp.int32),
            pltpu.VMEM((512,), jnp.int32),
            pltpu.VMEM((512,), jnp.int32),
            pltpu.VMEM((32,), jnp.int32),
            pltpu.VMEM((32,), jnp.int32),
            pltpu.VMEM((112,), f32),
            pltpu.VMEM((112,), f32),
            pltpu.VMEM((112,), f32),
            pltpu.VMEM((112,), f32),
            pltpu.VMEM((112,), f32),
            pltpu.VMEM((_MAX_OUT * 16,), f32),
        ],
    )(_scan_body)
    det = scan_k(us, gs, sy1a, sx1a, sy2a, sx2a)

    return det.reshape(_MAX_OUT, 16)[:, :5].reshape(B, _MAX_OUT, 5)


# speedup vs baseline: 1.0817x; 1.0068x over previous
import functools

import jax
import jax.numpy as jnp
from jax import lax
from jax.experimental import pallas as pl
from jax.experimental.pallas import tpu as pltpu
from jax.experimental.pallas import tpu_sc as plsc

_BBOX_STD = (0.1, 0.1, 0.2, 0.2)
_MAX_OUT = 100
_NMS_THR = 0.3
_CONF = 0.15
_NEG = -1e30

_ROWS = 160
_LANES = 128
_P = _ROWS * _LANES
_NT = 32
_CH = _P // _NT
_NC = 16
_D = _CH // _NC
_SORT_N = 64
_BIGI = jnp.int32(2**30)


def _prep_body(boxes_ref, deltas_ref, probs_ref, meta_ref,
               ukey_ref, y1_ref, x1_ref, y2_ref, x2_ref):
    h = meta_ref[0, 4]
    w = meta_ref[0, 5]
    wy1 = (meta_ref[0, 7] - 0.0) / (h - 1.0)
    wx1 = (meta_ref[0, 8] - 0.0) / (w - 1.0)
    wy2 = (meta_ref[0, 9] - 1.0) / (h - 1.0)
    wx2 = (meta_ref[0, 10] - 1.0) / (w - 1.0)

    ry1 = boxes_ref[0]
    rx1 = boxes_ref[1]
    ry2 = boxes_ref[2]
    rx2 = boxes_ref[3]
    dy = deltas_ref[0] * _BBOX_STD[0]
    dx = deltas_ref[1] * _BBOX_STD[1]
    dh = deltas_ref[2] * _BBOX_STD[2]
    dw = deltas_ref[3] * _BBOX_STD[3]

    height = ry2 - ry1
    width = rx2 - rx1
    cy = ry1 + 0.5 * height + dy * height
    cx = rx1 + 0.5 * width + dx * width
    height = height * jnp.exp(dh)
    width = width * jnp.exp(dw)
    y1 = cy - 0.5 * height
    x1 = cx - 0.5 * width
    y2 = y1 + height
    x2 = x1 + width

    y1_ref[...] = jnp.clip(y1, wy1, wy2)
    x1_ref[...] = jnp.clip(x1, wx1, wx2)
    y2_ref[...] = jnp.clip(y2, wy1, wy2)
    x2_ref[...] = jnp.clip(x2, wx1, wx2)

    probs = probs_ref[...]
    bits = lax.bitcast_convert_type(probs, jnp.int32)
    ukey_ref[...] = jnp.where(probs >= _CONF, bits, jnp.int32(-1))


def _lane():
    return lax.broadcasted_iota(jnp.int32, (16,), 0)


def _sort_body(ukey_hbm, y1_hbm, x1_hbm, y2_hbm, x2_hbm,
               us_hbm, gs_hbm, sy1_hbm, sx1_hbm, sy2_hbm, sx2_hbm,
               uloc, y1l, x1l, y2l, x2l, key, gid,
               e_us, e_gs, e_y1, e_x1, e_y2, e_x2):
    tid = lax.axis_index("s") * 2 + lax.axis_index("c")
    base = tid * _CH
    LANE = _lane()

    pltpu.sync_copy(ukey_hbm.at[pl.ds(base, _CH)], uloc)
    pltpu.sync_copy(y1_hbm.at[pl.ds(base, _CH)], y1l)
    pltpu.sync_copy(x1_hbm.at[pl.ds(base, _CH)], x1l)
    pltpu.sync_copy(y2_hbm.at[pl.ds(base, _CH)], y2l)
    pltpu.sync_copy(x2_hbm.at[pl.ds(base, _CH)], x2l)

    for r in range(_SORT_N):
        if r < _D:
            kv = uloc[pl.ds(r * 16, 16)]
            gv = base + r * 16 + LANE
        else:
            kv = jnp.full((16,), -1, jnp.int32)
            gv = jnp.full((16,), _P, jnp.int32)
        key[pl.ds(r * 16, 16)] = kv
        gid[pl.ds(r * 16, 16)] = gv

    k = 2
    while k <= _SORT_N:
        j = k // 2
        while j >= 1:
            sh = j.bit_length() - 1

            def ce4(i0, _, j=j, k=k, sh=sh):
                for q in range(4):
                    i = i0 * 4 + q
                    a = ((i >> sh) << (sh + 1)) | (i & (j - 1))
                    b = a | j
                    ka = key[pl.ds(a * 16, 16)]
                    kb = key[pl.ds(b * 16, 16)]
                    ga = gid[pl.ds(a * 16, 16)]
                    gb = gid[pl.ds(b * 16, 16)]
                    agtb = (ka > kb) | ((ka == kb) & (ga < gb))
                    kg = jnp.where(agtb, ka, kb)
                    kl = jnp.where(agtb, kb, ka)
                    gg = jnp.where(agtb, ga, gb)
                    gl = jnp.where(agtb, gb, ga)
                    desc = (a & k) == 0
                    key[pl.ds(a * 16, 16)] = jnp.where(desc, kg, kl)
                    key[pl.ds(b * 16, 16)] = jnp.where(desc, kl, kg)
                    gid[pl.ds(a * 16, 16)] = jnp.where(desc, gg, gl)
                    gid[pl.ds(b * 16, 16)] = jnp.where(desc, gl, gg)
                return 0

            lax.fori_loop(0, _SORT_N // 8, ce4, 0)
            j //= 2
        k *= 2

    for c in range(_NC):
        for ch in range((_D + 15) // 16):
            dvec = ch * 16 + LANE
            msk = dvec < _D
            src = jnp.minimum(dvec, _SORT_N - 1) * 16 + c
            kv = plsc.load_gather(key, [src])
            gv = plsc.load_gather(gid, [src])
            lidx = jnp.clip(gv - base, 0, _CH - 1)
            dst = jnp.minimum(c * _D + dvec, _CH - 1)
            plsc.store_scatter(e_us, [dst], kv, mask=msk)
            plsc.store_scatter(e_gs, [dst], gv, mask=msk)
            plsc.store_scatter(e_y1, [dst], plsc.load_gather(y1l, [lidx]),
                               mask=msk)
            plsc.store_scatter(e_x1, [dst], plsc.load_gather(x1l, [lidx]),
                               mask=msk)
            plsc.store_scatter(e_y2, [dst], plsc.load_gather(y2l, [lidx]),
                               mask=msk)
            plsc.store_scatter(e_x2, [dst], plsc.load_gather(x2l, [lidx]),
                               mask=msk)

    pltpu.sync_copy(e_us, us_hbm.at[pl.ds(base, _CH)])
    pltpu.sync_copy(e_gs, gs_hbm.at[pl.ds(base, _CH)])
    pltpu.sync_copy(e_y1, sy1_hbm.at[pl.ds(base, _CH)])
    pltpu.sync_copy(e_x1, sx1_hbm.at[pl.ds(base, _CH)])
    pltpu.sync_copy(e_y2, sy2_hbm.at[pl.ds(base, _CH)])
    pltpu.sync_copy(e_x2, sx2_hbm.at[pl.ds(base, _CH)])


def _splat_i(x):
    return jnp.zeros((16,), jnp.int32) + x


def _splat_f(x):
    return jnp.zeros((16,), jnp.float32) + x


def _scan_body(us_hbm, gs_hbm, sy1_hbm, sx1_hbm, sy2_hbm, sx2_hbm,
               det_hbm,
               su, sg, sy1, sx1, sy2, sx2,
               head_u, head_g, head_d, grp_u, grp_g,
               v_y1, v_x1, v_y2, v_x2, v_ar, outb):
    wid = lax.axis_index("s") * 2 + lax.axis_index("c")
    LANE = _lane()
    NS = _NT * _NC

    @pl.when(wid == 0)
    def _():
        pltpu.sync_copy(us_hbm, su)
        pltpu.sync_copy(gs_hbm, sg)
        pltpu.sync_copy(sy1_hbm, sy1)
        pltpu.sync_copy(sx1_hbm, sx1)
        pltpu.sync_copy(sy2_hbm, sy2)
        pltpu.sync_copy(sx2_hbm, sx2)

        zf = jnp.zeros((16,), jnp.float32)
        for jj in range(7):
            v_y1[pl.ds(jj * 16, 16)] = zf
            v_x1[pl.ds(jj * 16, 16)] = zf
            v_y2[pl.ds(jj * 16, 16)] = zf
            v_x2[pl.ds(jj * 16, 16)] = zf
            v_ar[pl.ds(jj * 16, 16)] = zf
        for r in range(_MAX_OUT):
            outb[pl.ds(r * 16, 16)] = zf

        def init_heads(i, _):
            svec = i * 16 + LANE
            posv = svec * _D
            hu = plsc.load_gather(su, [posv])
            hg = plsc.load_gather(sg, [posv])
            plsc.store_scatter(head_u, [svec], hu)
            plsc.store_scatter(head_g, [svec], hg)
            plsc.store_scatter(head_d, [svec], posv)
            gm = jnp.max(hu)
            gg = jnp.min(jnp.where(hu == gm, hg, _BIGI))
            m0 = LANE == 0
            plsc.store_scatter(grp_u, [_splat_i(i)], _splat_i(gm), mask=m0)
            plsc.store_scatter(grp_g, [_splat_i(i)], _splat_i(gg), mask=m0)
            return 0

        lax.fori_loop(0, NS // 16, init_heads, 0)

        def body(carry):
            count, _go = carry
            g0u = grp_u[pl.ds(0, 16)]
            g1u = grp_u[pl.ds(16, 16)]
            g0g = grp_g[pl.ds(0, 16)]
            g1g = grp_g[pl.ds(16, 16)]
            mu = jnp.max(jnp.maximum(g0u, g1u))
            valid = mu >= 0
            mg = jnp.min(jnp.minimum(
                jnp.where(g0u == mu, g0g, _BIGI),
                jnp.where(g1u == mu, g1g, _BIGI)))
            h0 = (g0u == mu) & (g0g == mg)
            h1 = (g1u == mu) & (g1g == mg)
            g = jnp.min(jnp.minimum(
                jnp.where(h0, LANE, jnp.int32(99)),
                jnp.where(h1, LANE + 16, jnp.int32(99))))
            g = jnp.minimum(g, jnp.int32(31))
            hu16 = head_u[pl.ds(g * 16, 16)]
            hg16 = head_g[pl.ds(g * 16, 16)]
            lsel = jnp.min(jnp.where((hu16 == mu) & (hg16 == mg), LANE,
                                     jnp.int32(15)))
            s = jnp.minimum(g * 16 + lsel, jnp.int32(NS - 1))
            sv_ = _splat_i(s)
            posv = jnp.minimum(plsc.load_gather(head_d, [sv_]),
                               jnp.int32(_P - 1))
            y1v = plsc.load_gather(sy1, [posv])
            x1v = plsc.load_gather(sx1, [posv])
            y2v = plsc.load_gather(sy2, [posv])
            x2v = plsc.load_gather(sx2, [posv])
            arv = (y2v - y1v) * (x2v - x1v)

            def iou_blk(jblk, acc):
                ya = v_y1[pl.ds(jblk * 16, 16)]
                xa = v_x1[pl.ds(jblk * 16, 16)]
                yb2 = v_y2[pl.ds(jblk * 16, 16)]
                xb2 = v_x2[pl.ds(jblk * 16, 16)]
                ar = v_ar[pl.ds(jblk * 16, 16)]
                yy1 = jnp.maximum(ya, y1v)
                xx1 = jnp.maximum(xa, x1v)
                yy2 = jnp.minimum(yb2, y2v)
                xx2 = jnp.minimum(xb2, x2v)
                inter = (jnp.maximum(yy2 - yy1, 0.0)
                         * jnp.maximum(xx2 - xx1, 0.0))
                denom = ar + arv - inter + 1e-9
                return jnp.maximum(acc, inter - _NMS_THR * denom)

            acc = lax.fori_loop(0, 7, iou_blk, jnp.full((16,), -1.0,
                                                        jnp.float32))
            sup = jnp.max(acc) > 0.0
            accept = valid & jnp.logical_not(sup)

            m0a = (LANE == 0) & accept
            cidx = _splat_i(count)
            plsc.store_scatter(v_y1, [cidx], y1v, mask=m0a)
            plsc.store_scatter(v_x1, [cidx], x1v, mask=m0a)
            plsc.store_scatter(v_y2, [cidx], y2v, mask=m0a)
            plsc.store_scatter(v_x2, [cidx], x2v, mask=m0a)
            plsc.store_scatter(v_ar, [cidx], arv, mask=m0a)

            scorev = plsc.bitcast(_splat_i(mu), jnp.float32)
            zfv = jnp.zeros((16,), jnp.float32)
            row = jnp.where(
                LANE == 0, y1v,
                jnp.where(LANE == 1, x1v,
                          jnp.where(LANE == 2, y2v,
                                    jnp.where(LANE == 3, x2v,
                                              jnp.where(LANE == 4, scorev,
                                                        zfv)))))
            outm = (jnp.zeros((16,), jnp.bool_) | accept)
            plsc.store_scatter(outb, [count * 16 + LANE], row, mask=outm)
            count2 = count + jnp.where(accept, 1, 0).astype(jnp.int32)

            nextv = posv + 1
            ex_v = (nextv % _D) == 0
            safe_next = jnp.minimum(nextv, jnp.int32(_P - 1))
            nu = jnp.where(ex_v, jnp.int32(-1),
                           plsc.load_gather(su, [safe_next]))
            ng = jnp.where(ex_v, _BIGI, plsc.load_gather(sg, [safe_next]))
            m0v = (LANE == 0) & valid
            plsc.store_scatter(head_u, [sv_], nu, mask=m0v)
            plsc.store_scatter(head_g, [sv_], ng, mask=m0v)
            plsc.store_scatter(head_d, [sv_], nextv, mask=m0v)
            hu2 = head_u[pl.ds(g * 16, 16)]
            hg2 = head_g[pl.ds(g * 16, 16)]
            gm2 = jnp.max(hu2)
            gg2 = jnp.min(jnp.where(hu2 == gm2, hg2, _BIGI))
            gidxv = _splat_i(g)
            plsc.store_scatter(grp_u, [gidxv], _splat_i(gm2), mask=m0v)
            plsc.store_scatter(grp_g, [gidxv], _splat_i(gg2), mask=m0v)

            go2 = valid & (count2 < _MAX_OUT)
            return (count2, go2)

        lax.while_loop(lambda c: c[1], body,
                       (jnp.int32(0), jnp.bool_(True)))

        pltpu.sync_copy(outb, det_hbm)


@functools.partial(jax.jit, static_argnames=())
def kernel(rois, bbox_scores, macacnn_bbox, image_meta):
    B, N = rois.shape[0], rois.shape[1]
    pad = _P - N

    def prep(x, fill):
        xt = jnp.transpose(x, (1, 0))
        xt = jnp.pad(xt, ((0, 0), (0, pad)), constant_values=fill)
        return xt.reshape(x.shape[1], _ROWS, _LANES)

    boxes_in = prep(rois[0], 0.0)
    deltas_in = prep(macacnn_bbox[0], 0.0)
    probs_in = prep(bbox_scores[0], -1.0)[0]

    f32 = jnp.float32
    ukey2d, y1p, x1p, y2p, x2p = pl.pallas_call(
        _prep_body,
        out_shape=[
            jax.ShapeDtypeStruct((_ROWS, _LANES), jnp.int32),
            jax.ShapeDtypeStruct((_ROWS, _LANES), f32),
            jax.ShapeDtypeStruct((_ROWS, _LANES), f32),
            jax.ShapeDtypeStruct((_ROWS, _LANES), f32),
            jax.ShapeDtypeStruct((_ROWS, _LANES), f32),
        ],
        in_specs=[
            pl.BlockSpec(memory_space=pltpu.VMEM),
            pl.BlockSpec(memory_space=pltpu.VMEM),
            pl.BlockSpec(memory_space=pltpu.VMEM),
            pl.BlockSpec(memory_space=pltpu.SMEM),
        ],
        out_specs=[pl.BlockSpec(memory_space=pltpu.VMEM)] * 5,
    )(boxes_in, deltas_in, probs_in, image_meta)

    ukey = ukey2d.reshape(_P)
    y1f = y1p.reshape(_P)
    x1f = x1p.reshape(_P)
    y2f = y2p.reshape(_P)
    x2f = x2p.reshape(_P)

    mesh = plsc.VectorSubcoreMesh(core_axis_name="c", subcore_axis_name="s")

    sc_params = pltpu.CompilerParams(needs_layout_passes=False)

    sort_k = functools.partial(
        pl.kernel, mesh=mesh, compiler_params=sc_params,
        out_type=[
            jax.ShapeDtypeStruct((_P,), jnp.int32),
            jax.ShapeDtypeStruct((_P,), jnp.int32),
            jax.ShapeDtypeStruct((_P,), f32),
            jax.ShapeDtypeStruct((_P,), f32),
            jax.ShapeDtypeStruct((_P,), f32),
            jax.ShapeDtypeStruct((_P,), f32),
        ],
        scratch_types=[
            pltpu.VMEM((_CH,), jnp.int32),
            pltpu.VMEM((_CH,), f32),
            pltpu.VMEM((_CH,), f32),
            pltpu.VMEM((_CH,), f32),
            pltpu.VMEM((_CH,), f32),
            pltpu.VMEM((_SORT_N * 16,), jnp.int32),
            pltpu.VMEM((_SORT_N * 16,), jnp.int32),
            pltpu.VMEM((_CH,), jnp.int32),
            pltpu.VMEM((_CH,), jnp.int32),
            pltpu.VMEM((_CH,), f32),
            pltpu.VMEM((_CH,), f32),
            pltpu.VMEM((_CH,), f32),
            pltpu.VMEM((_CH,), f32),
        ],
    )(_sort_body)
    us, gs, sy1a, sx1a, sy2a, sx2a = sort_k(ukey, y1f, x1f, y2f, x2f)

    scan_k = functools.partial(
        pl.kernel, mesh=mesh, compiler_params=sc_params,
        out_type=jax.ShapeDtypeStruct((_MAX_OUT * 16,), f32),
        scratch_types=[
            pltpu.VMEM((_P,), jnp.int32),
            pltpu.VMEM((_P,), jnp.int32),
            pltpu.VMEM((_P,), f32),
            pltpu.VMEM((_P,), f32),
            pltpu.VMEM((_P,), f32),
            pltpu.VMEM((_P,), f32),
            pltpu.VMEM((512,), jnp.int32),
            pltpu.VMEM((512,), jnp.int32),
            pltpu.VMEM((512,), jnp.int32),
            pltpu.VMEM((32,), jnp.int32),
            pltpu.VMEM((32,), jnp.int32),
            pltpu.VMEM((112,), f32),
            pltpu.VMEM((112,), f32),
            pltpu.VMEM((112,), f32),
            pltpu.VMEM((112,), f32),
            pltpu.VMEM((112,), f32),
            pltpu.VMEM((_MAX_OUT * 16,), f32),
        ],
    )(_scan_body)
    det = scan_k(us, gs, sy1a, sx1a, sy2a, sx2a)

    return det.reshape(_MAX_OUT, 16)[:, :5].reshape(B, _MAX_OUT, 5)
